# baseline (speedup 1.0000x reference)
import functools

import jax
import jax.numpy as jnp
from jax import lax
from jax.experimental import pallas as pl
from jax.experimental.pallas import tpu as pltpu
from jax.experimental.pallas import tpu_sc as plsc

N = 10000
E = 320000
D_IN = 128
H = 64

G = 16
C = H // G
EH = E // 2
IC = 4000
NCH = EH // IC
UR = 16

_DN = (((0,), (1,)), ((), ()))
_DT = (((0,), (0,)), ((), ()))


def _sc_segment_sum(hwT, ei, zeros):
    mesh = plsc.VectorSubcoreMesh(core_axis_name="c", subcore_axis_name="s")

    @functools.partial(
        pl.kernel,
        mesh=mesh,
        out_type=jax.ShapeDtypeStruct((2, H, N), jnp.float32),
        scratch_types=[
            pltpu.VMEM((C, N), jnp.float32),
            pltpu.VMEM((C, N), jnp.float32),
            pltpu.VMEM((IC,), jnp.int32),
            pltpu.VMEM((IC,), jnp.int32),
            pltpu.VMEM((IC,), jnp.int32),
            pltpu.VMEM((IC,), jnp.int32),
            pltpu.SemaphoreType.DMA,
            pltpu.SemaphoreType.DMA,
        ],
        compiler_params=pltpu.CompilerParams(use_tc_tiling_on_sc=False,
                                             needs_layout_passes=False),
    )
    def k(hwT_hbm, ei_hbm, zeros_hbm, out_hbm,
          hwg_v, accg_v, s0, d0, s1, d1, sem0, sem1):
        c = lax.axis_index("c")
        s = lax.axis_index("s")
        ebase = c * EH
        pltpu.sync_copy(hwT_hbm.at[pl.ds(C * s, C)], hwg_v)
        pltpu.sync_copy(zeros_hbm, accg_v)

        def fire(ch, sbuf, dbuf, sem):
            off = ebase + ch * IC
            pltpu.async_copy(ei_hbm.at[0].at[pl.ds(off, IC)], sbuf, sem)
            pltpu.async_copy(ei_hbm.at[1].at[pl.ds(off, IC)], dbuf, sem)

        def drain(ch, sbuf, dbuf, sem):
            off = ebase + ch * IC
            pltpu.make_async_copy(ei_hbm.at[0].at[pl.ds(off, IC)], sbuf, sem).wait()
            pltpu.make_async_copy(ei_hbm.at[1].at[pl.ds(off, IC)], dbuf, sem).wait()

        cols = [jnp.full((16,), cc, jnp.int32) for cc in range(C)]

        def inner(sbuf, dbuf):
            def blk(b):
                off = b * 16
                sv = sbuf[pl.ds(off, 16)]
                dv = dbuf[pl.ds(off, 16)]
                for cc in range(C):
                    vals = plsc.load_gather(hwg_v, [cols[cc], sv])
                    plsc.addupdate_scatter(accg_v, [cols[cc], dv], vals)
            plsc.parallel_loop(0, IC // 16, 1, unroll=UR)(blk)

        fire(0, s0, d0, sem0)
        fire(1, s1, d1, sem1)

        def body(g2, carry):
            ch = g2 * 2
            drain(ch, s0, d0, sem0)
            inner(s0, d0)

            @pl.when(ch + 2 < NCH)
            def _():
                fire(ch + 2, s0, d0, sem0)

            drain(ch + 1, s1, d1, sem1)
            inner(s1, d1)

            @pl.when(ch + 3 < NCH)
            def _():
                fire(ch + 3, s1, d1, sem1)

            return carry

        lax.fori_loop(0, NCH // 2, body, 0)
        pltpu.sync_copy(accg_v, out_hbm.at[c].at[pl.ds(C * s, C)])

    return k(hwT, ei, zeros)


def _head_body(h_ref, W_ref, Wr_ref, br_ref, hwT_ref, resT_ref):
    h = h_ref[...]
    hwT_ref[...] = lax.dot_general(W_ref[...], h, _DN,
                                   preferred_element_type=jnp.float32)
    r = lax.dot_general(Wr_ref[...], h, _DN,
                        preferred_element_type=jnp.float32)
    resT_ref[...] = jnp.maximum(r + br_ref[...][:, None], 0.0)


def _head(h, W, Wr, br):
    return pl.pallas_call(
        _head_body,
        out_shape=(jax.ShapeDtypeStruct((H, N), jnp.float32),
                   jax.ShapeDtypeStruct((H, N), jnp.float32)),
    )(h, W, Wr, br)


def _bn(parts_ref, resT_ref, b_ref, g_ref, bt_ref):
    aggT = parts_ref[0] + parts_ref[1]
    newT = jnp.maximum(aggT + b_ref[...][:, None], 0.0) + resT_ref[...]
    mean = jnp.mean(newT, axis=1, keepdims=True)
    var = jnp.mean((newT - mean) ** 2, axis=1, keepdims=True)
    return ((newT - mean) * lax.rsqrt(var + 1e-5) * g_ref[...][:, None]
            + bt_ref[...][:, None])


def _mid_body(parts_ref, resT_ref, b_ref, g_ref, bt_ref,
              W2_ref, Wr2_ref, br2_ref, hw2T_ref, res2T_ref):
    h1T = _bn(parts_ref, resT_ref, b_ref, g_ref, bt_ref)
    hw2T_ref[...] = lax.dot_general(W2_ref[...], h1T, _DT,
                                    preferred_element_type=jnp.float32)
    r = lax.dot_general(Wr2_ref[...], h1T, _DT,
                        preferred_element_type=jnp.float32)
    res2T_ref[...] = jnp.maximum(r + br2_ref[...][:, None], 0.0)


def _mid(parts, resT, b, g, bt, W2, Wr2, br2):
    return pl.pallas_call(
        _mid_body,
        out_shape=(jax.ShapeDtypeStruct((H, N), jnp.float32),
                   jax.ShapeDtypeStruct((H, N), jnp.float32)),
    )(parts, resT, b, g, bt, W2, Wr2, br2)


def _tail_body(parts_ref, resT_ref, b_ref, g_ref, bt_ref, out_ref):
    out_ref[...] = _bn(parts_ref, resT_ref, b_ref, g_ref, bt_ref)


def _tail(parts, resT, b, g, bt):
    return pl.pallas_call(
        _tail_body,
        out_shape=jax.ShapeDtypeStruct((H, N), jnp.float32),
    )(parts, resT, b, g, bt)


def kernel(feats, edge_index, W1, b1, Wr1, br1, g1, bt1,
           W2, b2, Wr2, br2, g2, bt2):
    ei = edge_index.astype(jnp.int32)
    zeros = jnp.zeros((C, N), jnp.float32)

    hw1T, res1T = _head(feats, W1, Wr1, br1)
    parts1 = _sc_segment_sum(hw1T, ei, zeros)
    hw2T, res2T = _mid(parts1, res1T, b1, g1, bt1, W2, Wr2, br2)
    parts2 = _sc_segment_sum(hw2T, ei, zeros)
    return _tail(parts2, res2T, b2, g2, bt2).T


# speedup vs baseline: 1.1269x; 1.1269x over previous
import functools

import jax
import jax.numpy as jnp
from jax import lax
from jax.experimental import pallas as pl
from jax.experimental.pallas import tpu as pltpu
from jax.experimental.pallas import tpu_sc as plsc

N = 10000
E = 320000
D_IN = 128
H = 64

G = 16
C = H // G
EH = E // 2
IC = 4000
NCH = EH // IC
UR = 8

_DN = (((0,), (1,)), ((), ()))
_DT = (((0,), (0,)), ((), ()))


def _sc_segment_sum(hwT, ei, zeros):
    mesh = plsc.VectorSubcoreMesh(core_axis_name="c", subcore_axis_name="s")

    @functools.partial(
        pl.kernel,
        mesh=mesh,
        out_type=jax.ShapeDtypeStruct((2, H, N), jnp.float32),
        scratch_types=[
            pltpu.VMEM((C, N), jnp.float32),
            pltpu.VMEM((C, N), jnp.float32),
            pltpu.VMEM((IC,), jnp.int32),
            pltpu.VMEM((IC,), jnp.int32),
            pltpu.VMEM((IC,), jnp.int32),
            pltpu.VMEM((IC,), jnp.int32),
            pltpu.SemaphoreType.DMA,
            pltpu.SemaphoreType.DMA,
        ],
        compiler_params=pltpu.CompilerParams(use_tc_tiling_on_sc=False,
                                             needs_layout_passes=False),
    )
    def k(hwT_hbm, ei_hbm, zeros_hbm, out_hbm,
          hwg_v, accg_v, s0, d0, s1, d1, sem0, sem1):
        c = lax.axis_index("c")
        s = lax.axis_index("s")
        ebase = c * EH
        pltpu.sync_copy(hwT_hbm.at[pl.ds(C * s, C)], hwg_v)
        pltpu.sync_copy(zeros_hbm, accg_v)

        def fire(ch, sbuf, dbuf, sem):
            off = ebase + ch * IC
            pltpu.async_copy(ei_hbm.at[0].at[pl.ds(off, IC)], sbuf, sem)
            pltpu.async_copy(ei_hbm.at[1].at[pl.ds(off, IC)], dbuf, sem)

        def drain(ch, sbuf, dbuf, sem):
            off = ebase + ch * IC
            pltpu.make_async_copy(ei_hbm.at[0].at[pl.ds(off, IC)], sbuf, sem).wait()
            pltpu.make_async_copy(ei_hbm.at[1].at[pl.ds(off, IC)], dbuf, sem).wait()

        cols = [jnp.full((16,), cc, jnp.int32) for cc in range(C)]

        def inner(sbuf, dbuf):
            def blk(b):
                off = b * 16
                sv = sbuf[pl.ds(off, 16)]
                dv = dbuf[pl.ds(off, 16)]
                for cc in range(C):
                    vals = plsc.load_gather(hwg_v, [cols[cc], sv])
                    plsc.addupdate_scatter(accg_v, [cols[cc], dv], vals)
            plsc.parallel_loop(0, IC // 16, 1, unroll=UR)(blk)

        fire(0, s0, d0, sem0)
        fire(1, s1, d1, sem1)

        def body(g2, carry):
            ch = g2 * 2
            drain(ch, s0, d0, sem0)
            inner(s0, d0)

            @pl.when(ch + 2 < NCH)
            def _():
                fire(ch + 2, s0, d0, sem0)

            drain(ch + 1, s1, d1, sem1)
            inner(s1, d1)

            @pl.when(ch + 3 < NCH)
            def _():
                fire(ch + 3, s1, d1, sem1)

            return carry

        lax.fori_loop(0, NCH // 2, body, 0)
        pltpu.sync_copy(accg_v, out_hbm.at[c].at[pl.ds(C * s, C)])

    return k(hwT, ei, zeros)


def _head_body(h_ref, W_ref, Wr_ref, br_ref, hwT_ref, resT_ref):
    h = h_ref[...]
    hwT_ref[...] = lax.dot_general(W_ref[...], h, _DN,
                                   preferred_element_type=jnp.float32)
    r = lax.dot_general(Wr_ref[...], h, _DN,
                        preferred_element_type=jnp.float32)
    resT_ref[...] = jnp.maximum(r + br_ref[...][:, None], 0.0)


def _head(h, W, Wr, br):
    return pl.pallas_call(
        _head_body,
        out_shape=(jax.ShapeDtypeStruct((H, N), jnp.float32),
                   jax.ShapeDtypeStruct((H, N), jnp.float32)),
    )(h, W, Wr, br)


def _bn(parts_ref, resT_ref, b_ref, g_ref, bt_ref):
    aggT = parts_ref[0] + parts_ref[1]
    newT = jnp.maximum(aggT + b_ref[...][:, None], 0.0) + resT_ref[...]
    mean = jnp.mean(newT, axis=1, keepdims=True)
    var = jnp.mean((newT - mean) ** 2, axis=1, keepdims=True)
    return ((newT - mean) * lax.rsqrt(var + 1e-5) * g_ref[...][:, None]
            + bt_ref[...][:, None])


def _mid_body(parts_ref, resT_ref, b_ref, g_ref, bt_ref,
              W2_ref, Wr2_ref, br2_ref, hw2T_ref, res2T_ref):
    h1T = _bn(parts_ref, resT_ref, b_ref, g_ref, bt_ref)
    hw2T_ref[...] = lax.dot_general(W2_ref[...], h1T, _DT,
                                    preferred_element_type=jnp.float32)
    r = lax.dot_general(Wr2_ref[...], h1T, _DT,
                        preferred_element_type=jnp.float32)
    res2T_ref[...] = jnp.maximum(r + br2_ref[...][:, None], 0.0)


def _mid(parts, resT, b, g, bt, W2, Wr2, br2):
    return pl.pallas_call(
        _mid_body,
        out_shape=(jax.ShapeDtypeStruct((H, N), jnp.float32),
                   jax.ShapeDtypeStruct((H, N), jnp.float32)),
    )(parts, resT, b, g, bt, W2, Wr2, br2)


def _tail_body(parts_ref, resT_ref, b_ref, g_ref, bt_ref, out_ref):
    out_ref[...] = _bn(parts_ref, resT_ref, b_ref, g_ref, bt_ref)


def _tail(parts, resT, b, g, bt):
    return pl.pallas_call(
        _tail_body,
        out_shape=jax.ShapeDtypeStruct((H, N), jnp.float32),
    )(parts, resT, b, g, bt)


def kernel(feats, edge_index, W1, b1, Wr1, br1, g1, bt1,
           W2, b2, Wr2, br2, g2, bt2):
    ei = edge_index.astype(jnp.int32)
    zeros = jnp.zeros((C, N), jnp.float32)

    hw1T, res1T = _head(feats, W1, Wr1, br1)
    parts1 = _sc_segment_sum(hw1T, ei, zeros)
    hw2T, res2T = _mid(parts1, res1T, b1, g1, bt1, W2, Wr2, br2)
    parts2 = _sc_segment_sum(hw2T, ei, zeros)
    return _tail(parts2, res2T, b2, g2, bt2).T
